# initial kernel scaffold (unmeasured)
import jax
import jax.numpy as jnp
from jax import lax
from jax.experimental import pallas as pl
from jax.experimental.pallas import tpu as pltpu


def kernel(
    x,
):
    def body(*refs):
        pass

    out_shape = jax.ShapeDtypeStruct(..., jnp.float32)
    return pl.pallas_call(body, out_shape=out_shape)(...)


# baseline (device time: 17863 ns/iter reference)
import functools

import jax
import jax.numpy as jnp
from jax import lax
from jax.experimental import pallas as pl
from jax.experimental.pallas import tpu as pltpu

N_DEV = 16


def kernel(x):
    m, n = x.shape

    def body(x_ref, out_ref, comm_ref, send_buf, send_sems, recv_sems):
        my = lax.axis_index("i")

        acc = x_ref[:, :]
        row = lax.broadcasted_iota(jnp.int32, (m, n), 0)
        shift = 1
        while shift < m:
            rolled = pltpu.roll(acc, shift, 0)
            acc = acc * jnp.where(row >= shift, rolled, 1.0)
            shift *= 2

        send_buf[:, :] = acc[m - 8 :, :]
        comm_ref[:, :, :] = jnp.ones((N_DEV, 8, n), jnp.float32)

        barrier = pltpu.get_barrier_semaphore()
        for j in range(N_DEV):
            pl.semaphore_signal(
                barrier, inc=1,
                device_id=(j,), device_id_type=pl.DeviceIdType.MESH,
            )
        pl.semaphore_wait(barrier, N_DEV)

        for j in range(N_DEV):
            @pl.when(my < j)
            def _():
                rdma = pltpu.make_async_remote_copy(
                    src_ref=send_buf,
                    dst_ref=comm_ref.at[my],
                    send_sem=send_sems.at[j],
                    recv_sem=recv_sems.at[my],
                    device_id=(j,),
                    device_id_type=pl.DeviceIdType.MESH,
                )
                rdma.start()

        for j in range(N_DEV):
            @pl.when(j < my)
            def _():
                recv = pltpu.make_async_remote_copy(
                    src_ref=send_buf,
                    dst_ref=comm_ref.at[j],
                    send_sem=send_sems.at[j],
                    recv_sem=recv_sems.at[j],
                    device_id=(j,),
                    device_id_type=pl.DeviceIdType.MESH,
                )
                recv.wait_recv()

        p = jnp.ones((1, n), jnp.float32)
        for j in range(N_DEV):
            slot = comm_ref[j]
            p = p * slot[7:8, :]
        out_ref[:, :] = acc * p

        for j in range(N_DEV):
            @pl.when(my < j)
            def _():
                rdma = pltpu.make_async_remote_copy(
                    src_ref=send_buf,
                    dst_ref=comm_ref.at[my],
                    send_sem=send_sems.at[j],
                    recv_sem=recv_sems.at[my],
                    device_id=(j,),
                    device_id_type=pl.DeviceIdType.MESH,
                )
                rdma.wait_send()

        @functools.partial(
            pl.run_scoped, sem2=pltpu.SemaphoreType.REGULAR
        )
        def _(sem2):
            for j in range(N_DEV):
                pl.semaphore_signal(
                    sem2, inc=1,
                    device_id=(j,), device_id_type=pl.DeviceIdType.MESH,
                )
            pl.semaphore_wait(sem2, N_DEV)

    return pl.pallas_call(
        body,
        out_shape=jax.ShapeDtypeStruct((m, n), jnp.float32),
        in_specs=[pl.BlockSpec(memory_space=pltpu.VMEM)],
        out_specs=pl.BlockSpec(memory_space=pltpu.VMEM),
        scratch_shapes=[
            pltpu.VMEM((N_DEV, 8, n), jnp.float32),
            pltpu.VMEM((8, n), jnp.float32),
            pltpu.SemaphoreType.DMA((N_DEV,)),
            pltpu.SemaphoreType.DMA((N_DEV,)),
        ],
        compiler_params=pltpu.CompilerParams(collective_id=0),
    )(x)
